# baseline (device time: 15744 ns/iter reference)
import jax
import jax.numpy as jnp
from jax import lax
from jax.experimental import pallas as pl
from jax.experimental.pallas import tpu as pltpu

N_DEV = 4
B = 2
SQ = 128
SKV = 128
D = 512
HQ = 8
DH = 64
SCALE = 0.125
BF = jnp.bfloat16


def kernel(x, Wq, Wo, K_ext, V_ext):
    pos = lax.axis_index("i")
    K2 = lax.dynamic_slice_in_dim(
        K_ext.reshape(B, SKV, 4 * HQ * DH), pos * HQ * DH, HQ * DH, axis=2
    ).astype(BF)
    V2 = lax.dynamic_slice_in_dim(
        V_ext.reshape(B, SKV, 4 * HQ * DH), pos * HQ * DH, HQ * DH, axis=2
    ).astype(BF)

    def body(x_ref, wq_ref, wo_ref, k_ref, v_ref, out_ref,
             attn_ref, comm_ref, send_sems, recv_sems):
        my_pos = lax.axis_index("i")

        barrier_sem = pltpu.get_barrier_semaphore()
        for d in range(1, N_DEV):
            peer = lax.rem(my_pos + d, N_DEV)
            pl.semaphore_signal(
                barrier_sem, inc=1,
                device_id=(peer,), device_id_type=pl.DeviceIdType.MESH,
            )

        wqb = wq_ref[...].astype(BF)
        q = lax.dot(x_ref[...].reshape(B * SQ, D).astype(BF), wqb,
                    preferred_element_type=jnp.float32).astype(BF)
        qs = (q[:SQ], q[SQ:])
        wob = wo_ref[...].astype(BF)

        chunk_rdmas = [[] for _ in range(B)]
        for b in range(B):
            for h in range(HQ):
                qbh = qs[b][:, h * DH:(h + 1) * DH]
                kbh = k_ref[b][:, h * DH:(h + 1) * DH]
                vbh = v_ref[b][:, h * DH:(h + 1) * DH]
                s = lax.dot_general(
                    qbh, kbh, (((1,), (1,)), ((), ())),
                    preferred_element_type=jnp.float32) * SCALE
                m = jnp.max(s, axis=1, keepdims=True)
                p = jnp.exp(s - m)
                l = jnp.sum(p, axis=1, keepdims=True)
                o = lax.dot(p.astype(BF), vbh,
                            preferred_element_type=jnp.float32)
                attn_ref[b * SQ:(b + 1) * SQ, h * DH:(h + 1) * DH] = (
                    (o / l).astype(BF))

            partial_b = lax.dot(
                attn_ref[b * SQ:(b + 1) * SQ, :], wob,
                preferred_element_type=jnp.float32)
            out_ref[b] = partial_b
            comm_ref[0, b] = partial_b.astype(BF)

            if b == 0:
                pl.semaphore_wait(barrier_sem, N_DEV - 1)

            for d in (2, 1, 3):
                peer = lax.rem(my_pos + d, N_DEV)
                rdma = pltpu.make_async_remote_copy(
                    src_ref=comm_ref.at[0, b],
                    dst_ref=comm_ref.at[d, b],
                    send_sem=send_sems.at[d - 1, b],
                    recv_sem=recv_sems.at[d - 1, b],
                    device_id=(peer,),
                    device_id_type=pl.DeviceIdType.MESH,
                )
                rdma.start()
                chunk_rdmas[b].append(rdma)

        for b in range(B):
            for rdma in chunk_rdmas[b]:
                rdma.wait_recv()
            out_ref[b] += (
                comm_ref[1, b].astype(jnp.float32)
                + comm_ref[2, b].astype(jnp.float32)
                + comm_ref[3, b].astype(jnp.float32)
            )
        for rdmas_b in chunk_rdmas:
            for rdma in rdmas_b:
                rdma.wait_send()

    return pl.pallas_call(
        body,
        out_shape=jax.ShapeDtypeStruct((B, SQ, D), jnp.float32),
        in_specs=[pl.BlockSpec(memory_space=pltpu.VMEM)] * 5,
        out_specs=pl.BlockSpec(memory_space=pltpu.VMEM),
        scratch_shapes=[
            pltpu.VMEM((B * SQ, HQ * DH), BF),
            pltpu.VMEM((N_DEV, B, SQ, D), BF),
            pltpu.SemaphoreType.DMA((N_DEV - 1, B)),
            pltpu.SemaphoreType.DMA((N_DEV - 1, B)),
        ],
        compiler_params=pltpu.CompilerParams(collective_id=0),
    )(x, Wq, Wo, K2, V2)


# device time: 15035 ns/iter; 1.0472x vs baseline; 1.0472x over previous
import jax
import jax.numpy as jnp
from jax import lax
from jax.experimental import pallas as pl
from jax.experimental.pallas import tpu as pltpu

N_DEV = 4
B = 2
SQ = 128
SKV = 128
D = 512
HQ = 8
DH = 64
SCALE = 0.125
BF = jnp.bfloat16


def kernel(x, Wq, Wo, K_ext, V_ext):
    pos = lax.axis_index("i")
    K2 = lax.dynamic_slice_in_dim(
        K_ext.reshape(B, SKV, 4 * HQ * DH), pos * HQ * DH, HQ * DH, axis=2
    ).astype(BF)
    V2 = lax.dynamic_slice_in_dim(
        V_ext.reshape(B, SKV, 4 * HQ * DH), pos * HQ * DH, HQ * DH, axis=2
    ).astype(BF)

    def body(x_ref, wq_ref, wo_ref, k_ref, v_ref, out_ref,
             attn_ref, comm_ref, send_sems, recv_sems):
        my_pos = lax.axis_index("i")

        barrier_sem = pltpu.get_barrier_semaphore()
        for d in range(1, N_DEV):
            peer = lax.rem(my_pos + d, N_DEV)
            pl.semaphore_signal(
                barrier_sem, inc=1,
                device_id=(peer,), device_id_type=pl.DeviceIdType.MESH,
            )

        wqb = wq_ref[...].astype(BF)
        q = lax.dot(x_ref[...].reshape(B * SQ, D).astype(BF), wqb,
                    preferred_element_type=jnp.float32)
        q = (q * SCALE).astype(BF)
        qs = (q[:SQ], q[SQ:])
        wob = wo_ref[...].astype(BF)

        chunk_rdmas = [[] for _ in range(B)]
        for b in range(B):
            for h in range(HQ):
                qbh = qs[b][:, h * DH:(h + 1) * DH]
                kbh = k_ref[b][:, h * DH:(h + 1) * DH]
                vbh = v_ref[b][:, h * DH:(h + 1) * DH]
                s = lax.dot_general(
                    qbh, kbh, (((1,), (1,)), ((), ())),
                    preferred_element_type=jnp.float32)
                p = jnp.exp(s)
                l = jnp.sum(p, axis=1, keepdims=True)
                o = lax.dot(p.astype(BF), vbh,
                            preferred_element_type=jnp.float32)
                attn_ref[b * SQ:(b + 1) * SQ, h * DH:(h + 1) * DH] = (
                    (o / l).astype(BF))

            partial_b = lax.dot(
                attn_ref[b * SQ:(b + 1) * SQ, :], wob,
                preferred_element_type=jnp.float32)
            out_ref[b] = partial_b
            comm_ref[0, b] = partial_b.astype(BF)

            if b == 0:
                pl.semaphore_wait(barrier_sem, N_DEV - 1)

            for d in range(1, N_DEV):
                peer = lax.rem(my_pos + d, N_DEV)
                rdma = pltpu.make_async_remote_copy(
                    src_ref=comm_ref.at[0, b],
                    dst_ref=comm_ref.at[d, b],
                    send_sem=send_sems.at[d - 1, b],
                    recv_sem=recv_sems.at[d - 1, b],
                    device_id=(peer,),
                    device_id_type=pl.DeviceIdType.MESH,
                )
                rdma.start()
                chunk_rdmas[b].append(rdma)

        for b in range(B):
            for rdma in chunk_rdmas[b]:
                rdma.wait_recv()
            out_ref[b] += (
                comm_ref[1, b].astype(jnp.float32)
                + comm_ref[2, b].astype(jnp.float32)
                + comm_ref[3, b].astype(jnp.float32)
            )
        for rdmas_b in chunk_rdmas:
            for rdma in rdmas_b:
                rdma.wait_send()

    return pl.pallas_call(
        body,
        out_shape=jax.ShapeDtypeStruct((B, SQ, D), jnp.float32),
        in_specs=[pl.BlockSpec(memory_space=pltpu.VMEM)] * 5,
        out_specs=pl.BlockSpec(memory_space=pltpu.VMEM),
        scratch_shapes=[
            pltpu.VMEM((B * SQ, HQ * DH), BF),
            pltpu.VMEM((N_DEV, B, SQ, D), BF),
            pltpu.SemaphoreType.DMA((N_DEV - 1, B)),
            pltpu.SemaphoreType.DMA((N_DEV - 1, B)),
        ],
        compiler_params=pltpu.CompilerParams(collective_id=0),
    )(x, Wq, Wo, K2, V2)


# device time: 13630 ns/iter; 1.1551x vs baseline; 1.1031x over previous
import jax
import jax.numpy as jnp
from jax import lax
from jax.experimental import pallas as pl
from jax.experimental.pallas import tpu as pltpu

N_DEV = 4
B = 2
SQ = 128
SKV = 128
D = 512
HQ = 8
DH = 64
SCALE = 0.125
BF = jnp.bfloat16


def kernel(x, Wq, Wo, K_ext, V_ext):
    pos = lax.axis_index("i")
    K2 = lax.dynamic_slice_in_dim(K_ext, pos * HQ, HQ, axis=2
        ).transpose(0, 2, 1, 3).astype(BF)
    V2 = lax.dynamic_slice_in_dim(V_ext, pos * HQ, HQ, axis=2
        ).transpose(0, 2, 1, 3).astype(BF)

    def body(x_ref, wq_ref, wo_ref, k_ref, v_ref, out_ref,
             attn_ref, comm_ref, send_sems, recv_sems):
        my_pos = lax.axis_index("i")

        barrier_sem = pltpu.get_barrier_semaphore()
        for d in range(1, N_DEV):
            peer = lax.rem(my_pos + d, N_DEV)
            pl.semaphore_signal(
                barrier_sem, inc=1,
                device_id=(peer,), device_id_type=pl.DeviceIdType.MESH,
            )

        wqb = wq_ref[...].astype(BF)
        q = lax.dot(x_ref[...].reshape(B * SQ, D).astype(BF), wqb,
                    preferred_element_type=jnp.float32)
        q = (q * SCALE).astype(BF)
        qs = (q[:SQ], q[SQ:])
        wob = wo_ref[...].astype(BF)

        chunk_rdmas = [[] for _ in range(B)]
        for b in range(B):
            q3 = jnp.transpose(
                qs[b].reshape(SQ, HQ, DH), (1, 0, 2))
            s = lax.dot_general(
                q3, k_ref[b], (((2,), (2,)), ((0,), (0,))),
                preferred_element_type=jnp.float32)
            p = jnp.exp(s)
            l = jnp.sum(p, axis=2, keepdims=True)
            o = lax.dot_general(
                p.astype(BF), v_ref[b], (((2,), (1,)), ((0,), (0,))),
                preferred_element_type=jnp.float32)
            o = o / l
            for h in range(HQ):
                attn_ref[b * SQ:(b + 1) * SQ, h * DH:(h + 1) * DH] = (
                    o[h].astype(BF))

            partial_b = lax.dot(
                attn_ref[b * SQ:(b + 1) * SQ, :], wob,
                preferred_element_type=jnp.float32)
            out_ref[b] = partial_b
            comm_ref[0, b] = partial_b.astype(BF)

            if b == 0:
                pl.semaphore_wait(barrier_sem, N_DEV - 1)

            for d in range(1, N_DEV):
                peer = lax.rem(my_pos + d, N_DEV)
                rdma = pltpu.make_async_remote_copy(
                    src_ref=comm_ref.at[0, b],
                    dst_ref=comm_ref.at[d, b],
                    send_sem=send_sems.at[d - 1, b],
                    recv_sem=recv_sems.at[d - 1, b],
                    device_id=(peer,),
                    device_id_type=pl.DeviceIdType.MESH,
                )
                rdma.start()
                chunk_rdmas[b].append(rdma)

        for b in range(B):
            for rdma in chunk_rdmas[b]:
                rdma.wait_recv()
            out_ref[b] += (
                comm_ref[1, b].astype(jnp.float32)
                + comm_ref[2, b].astype(jnp.float32)
                + comm_ref[3, b].astype(jnp.float32)
            )
        for rdmas_b in chunk_rdmas:
            for rdma in rdmas_b:
                rdma.wait_send()

    return pl.pallas_call(
        body,
        out_shape=jax.ShapeDtypeStruct((B, SQ, D), jnp.float32),
        in_specs=[pl.BlockSpec(memory_space=pltpu.VMEM)] * 5,
        out_specs=pl.BlockSpec(memory_space=pltpu.VMEM),
        scratch_shapes=[
            pltpu.VMEM((B * SQ, HQ * DH), BF),
            pltpu.VMEM((N_DEV, B, SQ, D), BF),
            pltpu.SemaphoreType.DMA((N_DEV - 1, B)),
            pltpu.SemaphoreType.DMA((N_DEV - 1, B)),
        ],
        compiler_params=pltpu.CompilerParams(collective_id=0),
    )(x, Wq, Wo, K2, V2)


# device time: 13622 ns/iter; 1.1558x vs baseline; 1.0006x over previous
import jax
import jax.numpy as jnp
from jax import lax
from jax.experimental import pallas as pl
from jax.experimental.pallas import tpu as pltpu

N_DEV = 4
B = 2
SQ = 128
SKV = 128
D = 512
HQ = 8
DH = 64
SCALE = 0.125
BF = jnp.bfloat16


def kernel(x, Wq, Wo, K_ext, V_ext):
    pos = lax.axis_index("i")
    K2 = lax.dynamic_slice_in_dim(K_ext, pos * HQ, HQ, axis=2
        ).transpose(0, 2, 1, 3).astype(BF)
    V2 = lax.dynamic_slice_in_dim(V_ext, pos * HQ, HQ, axis=2
        ).transpose(0, 2, 1, 3).astype(BF)

    def body(x_ref, wq_ref, wo_ref, k_ref, v_ref, out_ref,
             attn_ref, comm_ref, send_sems, recv_sems):
        my_pos = lax.axis_index("i")

        barrier_sem = pltpu.get_barrier_semaphore()
        for d in range(1, N_DEV):
            peer = lax.rem(my_pos + d, N_DEV)
            pl.semaphore_signal(
                barrier_sem, inc=1,
                device_id=(peer,), device_id_type=pl.DeviceIdType.MESH,
            )

        wqb = wq_ref[...].astype(BF)
        q = lax.dot(x_ref[...].reshape(B * SQ, D).astype(BF), wqb,
                    preferred_element_type=jnp.float32)
        q = (q * SCALE).astype(BF)
        qs = (q[:SQ], q[SQ:])
        wob = wo_ref[...].astype(BF)

        chunk_rdmas = [[] for _ in range(B)]
        for b in range(B):
            q3 = jnp.transpose(
                qs[b].reshape(SQ, HQ, DH), (1, 0, 2))
            s = lax.dot_general(
                q3, k_ref[b], (((2,), (2,)), ((0,), (0,))),
                preferred_element_type=jnp.float32)
            p = jnp.exp(s)
            l = jnp.sum(p, axis=2, keepdims=True)
            o = lax.dot_general(
                p.astype(BF), v_ref[b], (((2,), (1,)), ((0,), (0,))),
                preferred_element_type=jnp.float32)
            o = o / l
            attn_ref[b * SQ:(b + 1) * SQ, :] = jnp.transpose(
                o, (1, 0, 2)).reshape(SQ, HQ * DH).astype(BF)

            partial_b = lax.dot(
                attn_ref[b * SQ:(b + 1) * SQ, :], wob,
                preferred_element_type=jnp.float32)
            out_ref[b] = partial_b
            comm_ref[0, b] = partial_b.astype(BF)

            if b == 0:
                pl.semaphore_wait(barrier_sem, N_DEV - 1)

            for d in range(1, N_DEV):
                peer = lax.rem(my_pos + d, N_DEV)
                rdma = pltpu.make_async_remote_copy(
                    src_ref=comm_ref.at[0, b],
                    dst_ref=comm_ref.at[d, b],
                    send_sem=send_sems.at[d - 1, b],
                    recv_sem=recv_sems.at[d - 1, b],
                    device_id=(peer,),
                    device_id_type=pl.DeviceIdType.MESH,
                )
                rdma.start()
                chunk_rdmas[b].append(rdma)

        for b in range(B):
            for rdma in chunk_rdmas[b]:
                rdma.wait_recv()
            out_ref[b] += (
                comm_ref[1, b].astype(jnp.float32)
                + comm_ref[2, b].astype(jnp.float32)
                + comm_ref[3, b].astype(jnp.float32)
            )
        for rdmas_b in chunk_rdmas:
            for rdma in rdmas_b:
                rdma.wait_send()

    return pl.pallas_call(
        body,
        out_shape=jax.ShapeDtypeStruct((B, SQ, D), jnp.float32),
        in_specs=[pl.BlockSpec(memory_space=pltpu.VMEM)] * 5,
        out_specs=pl.BlockSpec(memory_space=pltpu.VMEM),
        scratch_shapes=[
            pltpu.VMEM((B * SQ, HQ * DH), BF),
            pltpu.VMEM((N_DEV, B, SQ, D), BF),
            pltpu.SemaphoreType.DMA((N_DEV - 1, B)),
            pltpu.SemaphoreType.DMA((N_DEV - 1, B)),
        ],
        compiler_params=pltpu.CompilerParams(collective_id=0),
    )(x, Wq, Wo, K2, V2)


# device time: 13619 ns/iter; 1.1560x vs baseline; 1.0002x over previous
import jax
import jax.numpy as jnp
from jax import lax
from jax.experimental import pallas as pl
from jax.experimental.pallas import tpu as pltpu

N_DEV = 4
B = 2
SQ = 128
SKV = 128
D = 512
HQ = 8
DH = 64
SCALE = 0.125
BF = jnp.bfloat16


def kernel(x, Wq, Wo, K_ext, V_ext):
    pos = lax.axis_index("i")
    K2 = lax.dynamic_slice_in_dim(K_ext, pos * HQ, HQ, axis=2
        ).transpose(0, 2, 1, 3).astype(BF)
    V2 = lax.dynamic_slice_in_dim(V_ext, pos * HQ, HQ, axis=2
        ).transpose(0, 2, 1, 3).astype(BF)

    def body(x_ref, wq_ref, wo_ref, k_ref, v_ref, out_ref,
             comm_ref, send_sems, recv_sems):
        my_pos = lax.axis_index("i")

        barrier_sem = pltpu.get_barrier_semaphore()
        for d in range(1, N_DEV):
            peer = lax.rem(my_pos + d, N_DEV)
            pl.semaphore_signal(
                barrier_sem, inc=1,
                device_id=(peer,), device_id_type=pl.DeviceIdType.MESH,
            )

        wqb = wq_ref[...].astype(BF)
        q = lax.dot(x_ref[...].reshape(B * SQ, D).astype(BF), wqb,
                    preferred_element_type=jnp.float32)
        q = (q * SCALE).astype(BF)
        qs = (q[:SQ], q[SQ:])
        wob = wo_ref[...].astype(BF)

        chunk_rdmas = [[] for _ in range(B)]
        for b in range(B):
            q3 = jnp.transpose(
                qs[b].reshape(SQ, HQ, DH), (1, 0, 2))
            s = lax.dot_general(
                q3, k_ref[b], (((2,), (2,)), ((0,), (0,))),
                preferred_element_type=jnp.float32)
            p = jnp.exp(s)
            l = jnp.sum(p, axis=2, keepdims=True)
            o = lax.dot_general(
                p.astype(BF), v_ref[b], (((2,), (1,)), ((0,), (0,))),
                preferred_element_type=jnp.float32)
            o = o / l
            attn_b = jnp.transpose(
                o, (1, 0, 2)).reshape(SQ, HQ * DH).astype(BF)

            partial_b = lax.dot(
                attn_b, wob,
                preferred_element_type=jnp.float32)
            out_ref[b] = partial_b
            comm_ref[0, b] = partial_b.astype(BF)

            if b == 0:
                pl.semaphore_wait(barrier_sem, N_DEV - 1)

            for d in range(1, N_DEV):
                peer = lax.rem(my_pos + d, N_DEV)
                rdma = pltpu.make_async_remote_copy(
                    src_ref=comm_ref.at[0, b],
                    dst_ref=comm_ref.at[d, b],
                    send_sem=send_sems.at[d - 1, b],
                    recv_sem=recv_sems.at[d - 1, b],
                    device_id=(peer,),
                    device_id_type=pl.DeviceIdType.MESH,
                )
                rdma.start()
                chunk_rdmas[b].append(rdma)

        for b in range(B):
            for rdma in chunk_rdmas[b]:
                rdma.wait_recv()
            out_ref[b] += (
                comm_ref[1, b].astype(jnp.float32)
                + comm_ref[2, b].astype(jnp.float32)
                + comm_ref[3, b].astype(jnp.float32)
            )
        for rdmas_b in chunk_rdmas:
            for rdma in rdmas_b:
                rdma.wait_send()

    return pl.pallas_call(
        body,
        out_shape=jax.ShapeDtypeStruct((B, SQ, D), jnp.float32),
        in_specs=[pl.BlockSpec(memory_space=pltpu.VMEM)] * 5,
        out_specs=pl.BlockSpec(memory_space=pltpu.VMEM),
        scratch_shapes=[
            pltpu.VMEM((N_DEV, B, SQ, D), BF),
            pltpu.SemaphoreType.DMA((N_DEV - 1, B)),
            pltpu.SemaphoreType.DMA((N_DEV - 1, B)),
        ],
        compiler_params=pltpu.CompilerParams(collective_id=0),
    )(x, Wq, Wo, K2, V2)
